# baseline (speedup 1.0000x reference)
import jax
import jax.numpy as jnp
from jax.experimental import pallas as pl
from jax.experimental.pallas import tpu as pltpu

_FEM_WEIGHT = 1.0
_REG_WEIGHT = 0.1


def _loss_kernel(nm_hbm, fm_hbm, pr_hbm, out_ref, nm_v, fm_v, pr_v, sems):
    cp_nm = pltpu.make_async_copy(nm_hbm, nm_v, sems.at[0])
    cp_fm = pltpu.make_async_copy(fm_hbm, fm_v, sems.at[1])
    cp_pr = pltpu.make_async_copy(pr_hbm, pr_v, sems.at[2])
    cp_nm.start()
    cp_fm.start()
    cp_pr.start()

    cp_nm.wait()
    cp_fm.wait()
    d = nm_v[...] - fm_v[...]
    fem = jnp.sum(d * d)

    cp_pr.wait()
    p = pr_v[...]
    core = p[:, :, :-1, :-1, :-1]
    dx = p[:, :, 1:, :-1, :-1] - core
    dy = p[:, :, :-1, 1:, :-1] - core
    dz = p[:, :, :-1, :-1, 1:] - core
    reg = jnp.sum(dx * dx) + jnp.sum(dy * dy) + jnp.sum(dz * dz)

    n_total = 1.0
    for s in nm_v.shape:
        n_total *= s
    n_bc = nm_v.shape[0] * nm_v.shape[1]
    out_ref[0, 0] = fem * (_FEM_WEIGHT / n_total) + reg * (_REG_WEIGHT / n_bc)


def kernel(network_mesh, pc, fem_mesh, pred):
    del pc
    shape = network_mesh.shape
    any_spec = pl.BlockSpec(memory_space=pl.ANY)
    out = pl.pallas_call(
        _loss_kernel,
        out_shape=jax.ShapeDtypeStruct((1, 1), jnp.float32),
        in_specs=[any_spec, any_spec, any_spec],
        out_specs=pl.BlockSpec(memory_space=pltpu.SMEM),
        scratch_shapes=[
            pltpu.VMEM(shape, jnp.float32),
            pltpu.VMEM(shape, jnp.float32),
            pltpu.VMEM(shape, jnp.float32),
            pltpu.SemaphoreType.DMA((3,)),
        ],
    )(network_mesh, fem_mesh, pred)
    return out[0, 0]
